# baseline (speedup 1.0000x reference)
import functools

import jax
import jax.numpy as jnp
from jax.experimental import pallas as pl


def _lrelu(v, s):
    return jnp.where(v >= 0, v, s * v)


def _pad_rows(a, m):
    r = a.shape[0]
    rp = ((r + m - 1) // m) * m
    if rp == r:
        return a
    pad = [(0, rp - r)] + [(0, 0)] * (a.ndim - 1)
    return jnp.pad(a, pad)


def _mm_plain_body(x_ref, w_ref, o_ref):
    o_ref[...] = jnp.dot(x_ref[...], w_ref[...],
                         preferred_element_type=jnp.float32)


def _mm_actbias_body(x_ref, w_ref, b_ref, o_ref):
    xa = _lrelu(x_ref[...] + b_ref[0:1, :], 0.01)
    o_ref[...] = jnp.dot(xa, w_ref[...], preferred_element_type=jnp.float32)


def _brow(b, k):
    return jnp.zeros((8, k), jnp.float32).at[0].set(b)


def _mm(x, w, bn=256, bias=None):
    xp = _pad_rows(x, bn)
    r, k = xp.shape
    m = w.shape[1]
    grid = (r // bn,)
    if bias is None:
        out = pl.pallas_call(
            _mm_plain_body,
            grid=grid,
            in_specs=[pl.BlockSpec((bn, k), lambda i: (i, 0)),
                      pl.BlockSpec((k, m), lambda i: (0, 0))],
            out_specs=pl.BlockSpec((bn, m), lambda i: (i, 0)),
            out_shape=jax.ShapeDtypeStruct((r, m), jnp.float32),
        )(xp, w)
    else:
        out = pl.pallas_call(
            _mm_actbias_body,
            grid=grid,
            in_specs=[pl.BlockSpec((bn, k), lambda i: (i, 0)),
                      pl.BlockSpec((k, m), lambda i: (0, 0)),
                      pl.BlockSpec((8, k), lambda i: (0, 0))],
            out_specs=pl.BlockSpec((bn, m), lambda i: (i, 0)),
            out_shape=jax.ShapeDtypeStruct((r, m), jnp.float32),
        )(xp, w, _brow(bias, k))
    return out[:x.shape[0]]


def _ew3_body(a_ref, b_ref, c_ref, o_ref):
    o_ref[...] = _lrelu(a_ref[...] + b_ref[...] + c_ref[...], 0.2)


def _exp_body(a_ref, m_ref, o_ref):
    o_ref[...] = jnp.exp(a_ref[...] - m_ref[...])


def _wscale_body(h_ref, e_ref, d_ref, o_ref):
    w = e_ref[:, 0:1] / (d_ref[:, 0:1] + 1e-16)
    o_ref[...] = h_ref[...] * w


def _sheets(v, bw=128, br=8):
    l = v.shape[0]
    rows = (l + bw - 1) // bw
    rp = ((rows + br - 1) // br) * br
    vp = jnp.pad(v, (0, rp * bw - l))
    return vp.reshape(rp, bw)


def _ew_sheets(body, args, bw=128, br=8):
    r = args[0].shape[0]
    grid = (r // br,)
    spec = pl.BlockSpec((br, bw), lambda i: (i, 0))
    return pl.pallas_call(
        body, grid=grid,
        in_specs=[spec] * len(args),
        out_specs=spec,
        out_shape=jax.ShapeDtypeStruct((r, bw), jnp.float32),
    )(*args)


def _edge_mlp_body(g1_ref, g2_ref, e3_ref, b1_ref, w2_ref, b2_ref,
                   tw1_ref, tb1_ref, vw1_ref, vb1_ref, wlast_ref, o_ref):
    c = _lrelu(g1_ref[...] + g2_ref[...] + e3_ref[...] + b1_ref[0:1, :], 0.01)
    c2 = _lrelu(jnp.dot(c, w2_ref[...], preferred_element_type=jnp.float32)
                + b2_ref[0:1, :], 0.01)
    t1 = _lrelu(jnp.dot(c2, tw1_ref[...], preferred_element_type=jnp.float32)
                + tb1_ref[0:1, :], 0.01)
    v1 = _lrelu(jnp.dot(c2, vw1_ref[...], preferred_element_type=jnp.float32)
                + vb1_ref[0:1, :], 0.01)
    tv2 = jnp.dot(jnp.concatenate([t1, v1], axis=1), wlast_ref[...],
                  preferred_element_type=jnp.float32)
    te = jax.nn.sigmoid(tv2[:, 0:1] + wlast_ref[0, 2])
    tv = tv2[:, 1:2] + wlast_ref[0, 3]
    o_ref[...] = jnp.concatenate(
        [te * tv, te, jnp.zeros_like(tv2[:, 2:8])], axis=1)


def _edge_mlp(g1s, g2s, e3, p, be=512):
    ep, hc4 = g1s.shape
    hc2, hc = hc4 // 2, hc4 // 4
    w2 = p['ef_w2'].T
    tw1 = p['tc_w1'].T
    vw1 = p['vr_w1'].T
    wlast = jnp.zeros((hc2, 8), jnp.float32)
    wlast = wlast.at[:hc, 0].set(p['tc_w2'][0])
    wlast = wlast.at[hc:, 1].set(p['vr_w2'][0])
    wlast = wlast.at[0, 2].set(p['tc_b2'][0])
    wlast = wlast.at[0, 3].set(p['vr_b2'][0])
    grid = (ep // be,)

    def bs(r, c):
        return pl.BlockSpec((r, c), lambda i: (i, 0))

    def const(r, c):
        return pl.BlockSpec((r, c), lambda i: (0, 0))

    return pl.pallas_call(
        _edge_mlp_body, grid=grid,
        in_specs=[bs(be, hc4), bs(be, hc4), bs(be, hc4),
                  const(8, hc4),
                  const(hc4, hc2),
                  const(8, hc2),
                  const(hc2, hc),
                  const(8, hc),
                  const(hc2, hc),
                  const(8, hc),
                  const(hc2, 8)],
        out_specs=bs(be, 8),
        out_shape=jax.ShapeDtypeStruct((ep, 8), jnp.float32),
    )(g1s, g2s, e3, _brow(p['ef_b1'], hc4), w2,
      _brow(p['ef_b2'], hc2), tw1, _brow(p['tc_b1'], hc),
      vw1, _brow(p['vr_b1'], hc), wlast)


def _gat_layer(x, src2, dst2, et, n, prev_b, W, a_src, a_dst, b):
    o = W.shape[0]
    ws = W.T @ a_src
    wd = W.T @ a_dst
    wt = jnp.concatenate(
        [W.T, ws[:, None], wd[:, None],
         jnp.zeros((W.shape[1], 6), jnp.float32)], axis=1)
    hsd = _mm(x, wt, bias=prev_b)
    h = hsd[:, :o]
    s = hsd[:, o]
    d = hsd[:, o + 1]

    sg = _sheets(s[src2])
    dg = _sheets(d[dst2])
    ets = _sheets(et)
    al = _ew_sheets(_ew3_body, (sg, dg, ets))
    al_flat = al.reshape(-1)[:src2.shape[0]]
    amax = jax.ops.segment_max(al_flat, dst2, num_segments=n)
    amax = jnp.where(jnp.isfinite(amax), amax, 0.0)
    ex = _ew_sheets(_exp_body, (al, _sheets(amax[dst2])))
    ex_flat = ex.reshape(-1)[:src2.shape[0]]
    den = jax.ops.segment_sum(ex_flat, dst2, num_segments=n)

    hs = _pad_rows(h[src2], 512)
    exc = _pad_rows(jnp.broadcast_to(ex_flat[:, None], (ex_flat.shape[0], 8)), 512)
    dng = den[dst2]
    dnc = _pad_rows(jnp.broadcast_to(dng[:, None], (dng.shape[0], 8)), 512)
    grid = (hs.shape[0] // 512,)
    hw = pl.pallas_call(
        _wscale_body, grid=grid,
        in_specs=[pl.BlockSpec((512, o), lambda i: (i, 0)),
                  pl.BlockSpec((512, 8), lambda i: (i, 0)),
                  pl.BlockSpec((512, 8), lambda i: (i, 0))],
        out_specs=pl.BlockSpec((512, o), lambda i: (i, 0)),
        out_shape=jax.ShapeDtypeStruct((hs.shape[0], o), jnp.float32),
    )(hs, exc, dnc)[:src2.shape[0]]
    return jax.ops.segment_sum(hw, dst2, num_segments=n)


def kernel(x, edge_index, edge_attr, params):
    p = params
    n = x.shape[0]
    e = edge_index.shape[1]
    src, dst = edge_index[0], edge_index[1]
    loop = jnp.arange(n, dtype=src.dtype)
    src2 = jnp.concatenate([src, loop])
    dst2 = jnp.concatenate([dst, loop])
    ea_mean = edge_attr.mean(axis=0)
    ea2 = jnp.concatenate(
        [edge_attr, jnp.broadcast_to(ea_mean, (n, edge_attr.shape[1]))], axis=0)

    et = [ea2 @ (p['We%d' % i].T @ p['ae%d' % i]) for i in (1, 2, 3)]

    agg1 = _gat_layer(x, src2, dst2, et[0], n, None,
                      p['W1'], p['as1'], p['ad1'], p['b1'])
    agg2 = _gat_layer(agg1, src2, dst2, et[1], n, p['b1'],
                      p['W2'], p['as2'], p['ad2'], p['b2'])
    agg3 = _gat_layer(agg2, src2, dst2, et[2], n, p['b2'],
                      p['W3'], p['as3'], p['ad3'], p['b3'])

    hc4 = p['ef_w1'].shape[0]
    de = edge_attr.shape[1]
    a1t = p['ef_w1'][:, :hc4].T
    a2t = p['ef_w1'][:, hc4:2 * hc4].T
    a3t = p['ef_w1'][:, 2 * hc4:].T
    g = _mm(agg3, jnp.concatenate([a1t, a2t], axis=1), bias=p['b3'])
    g1s = _pad_rows(g[src, :hc4], 512)
    g2s = _pad_rows(g[dst, hc4:], 512)
    e3 = _mm(_pad_rows(edge_attr, 512), a3t, bn=512)
    out = _edge_mlp(g1s, g2s, e3, p)[:e]
    return (out[:, 0:1], out[:, 1:2])
